# baseline (speedup 1.0000x reference)
import functools

import jax
import jax.numpy as jnp
from jax import lax
from jax.experimental import pallas as pl
from jax.experimental.pallas import tpu as pltpu
from jax.experimental.pallas import tpu_sc as plsc

NC = 2
NS = 16
NW = NC * NS
CH = 80
ZR = 128
WL = 128
RB = 5


def _make_agg(n, e):
    epw = e // NW
    n_ch = epw // CH
    rb = 3
    n_out = n_ch // rb
    rem = n_ch - n_out * rb
    npad = -(-n // (NS * ZR)) * (NS * ZR)
    rps = npad // NS
    mesh = plsc.VectorSubcoreMesh(core_axis_name="c", subcore_axis_name="s")

    @functools.partial(
        pl.kernel,
        mesh=mesh,
        out_type=jax.ShapeDtypeStruct((NC, npad, WL), jnp.float32),
        scratch_types=[
            pltpu.VMEM((n_ch, CH), jnp.int32),
            pltpu.VMEM((rb, CH), jnp.int32),
            pltpu.VMEM((rb, CH), jnp.int32),
            pltpu.VMEM((rb, CH, WL), jnp.float32),
            pltpu.VMEM_SHARED((npad, WL), jnp.float32),
            pltpu.SemaphoreType.DMA,
            pltpu.SemaphoreType.DMA,
            pltpu.SemaphoreType.DMA,
        ],
    )
    def agg(y_hbm, comb_hbm, zeros_hbm, out_hbm,
            comb_v, sidx_v, didx_v, rows_v, acc_sh, s0, s1, s2):
        sems = (s0, s1, s2)
        cid = lax.axis_index("c")
        sid = lax.axis_index("s")
        wid = sid * NC + cid

        def unpack_and_fire(g, b):
            for t in range(CH // 16):
                v = comb_v[g, pl.ds(t * 16, 16)]
                sidx_v[b, pl.ds(t * 16, 16)] = lax.bitwise_and(v, 0xFFFF)
                didx_v[b, pl.ds(t * 16, 16)] = lax.shift_right_logical(v, 16)
            pltpu.async_copy(y_hbm.at[sidx_v.at[b]], rows_v.at[b], sems[b])

        def drain_and_scatter(g, b):
            pltpu.make_async_copy(y_hbm.at[sidx_v.at[b]], rows_v.at[b],
                                  sems[b]).wait()
            pltpu.sync_copy(rows_v.at[b], acc_sh.at[didx_v.at[b]], add=True)

        pltpu.sync_copy(comb_hbm.at[wid], comb_v)
        pltpu.sync_copy(zeros_hbm, acc_sh.at[pl.ds(sid * rps, rps)])
        for b in range(rb):
            unpack_and_fire(b, b)
        plsc.subcore_barrier()

        def body(k, _):
            for b in range(rb):
                g = k * rb + b
                drain_and_scatter(g, b)

                @pl.when(g + rb < n_ch)
                def _():
                    unpack_and_fire(g + rb, b)
            return 0

        lax.fori_loop(0, n_out, body, 0)
        for j in range(rem):
            drain_and_scatter(n_out * rb + j, j)
        plsc.subcore_barrier()

        pltpu.sync_copy(acc_sh.at[pl.ds(sid * rps, rps)],
                        out_hbm.at[cid, pl.ds(sid * rps, rps)])

    return agg


def _make_edge_gather(n, e, w):
    epw = e // NW
    n_ch = epw // CH
    rb = 4
    n_out = n_ch // rb
    rem = n_ch - n_out * rb
    pkr = CH // 2
    mesh = plsc.VectorSubcoreMesh(core_axis_name="c", subcore_axis_name="s")

    @functools.partial(
        pl.kernel,
        mesh=mesh,
        out_type=jax.ShapeDtypeStruct((e // 2, WL), jnp.float32),
        scratch_types=[
            pltpu.VMEM((n_ch, CH), jnp.int32),
            pltpu.VMEM((rb, CH), jnp.int32),
            pltpu.VMEM((rb, CH), jnp.int32),
            pltpu.VMEM((rb, CH, WL), jnp.float32),
            pltpu.VMEM((rb, CH, WL), jnp.float32),
            pltpu.VMEM((rb // 2, 2 * pkr, WL), jnp.float32),
            pltpu.SemaphoreType.DMA,
            pltpu.SemaphoreType.DMA,
            pltpu.SemaphoreType.DMA,
            pltpu.SemaphoreType.DMA,
            pltpu.SemaphoreType.DMA,
            pltpu.SemaphoreType.DMA,
        ],
    )
    def gat(h_hbm, comb_hbm, out_hbm, comb_v, sidx_v, didx_v,
            srows_v, drows_v, pk_v, g0, g1, g2, g3, w0, w1):
        gsem = (g0, g1, g2, g3)
        wsem = (w0, w1)
        cid = lax.axis_index("c")
        sid = lax.axis_index("s")
        wid = sid * NC + cid

        def unpack_and_fire(g, b):
            for t in range(CH // 16):
                v = comb_v[g, pl.ds(t * 16, 16)]
                sidx_v[b, pl.ds(t * 16, 16)] = lax.bitwise_and(v, 0xFFFF)
                didx_v[b, pl.ds(t * 16, 16)] = lax.shift_right_logical(v, 16)
            pltpu.async_copy(h_hbm.at[sidx_v.at[b]], srows_v.at[b], gsem[b])
            pltpu.async_copy(h_hbm.at[didx_v.at[b]], drows_v.at[b], gsem[b])

        def wait_write(b, nrows):
            pltpu.make_async_copy(pk_v.at[b // 2, pl.ds(0, nrows)],
                                  out_hbm.at[pl.ds(0, nrows)], wsem[b // 2]
                                  ).wait()

        def drain_pack(g, b):
            pltpu.make_async_copy(h_hbm.at[sidx_v.at[b]], srows_v.at[b],
                                  gsem[b]).wait()
            pltpu.make_async_copy(h_hbm.at[didx_v.at[b]], drows_v.at[b],
                                  gsem[b]).wait()
            half = (b % 2) * pkr
            for j in range(pkr):
                for q, (ref, row) in enumerate(
                        ((srows_v, 2 * j), (drows_v, 2 * j),
                         (srows_v, 2 * j + 1), (drows_v, 2 * j + 1))):
                    for t in range(2):
                        pk_v[b // 2, half + j, pl.ds(32 * q + 16 * t, 16)] = (
                            ref[b, row, pl.ds(16 * t, 16)])

        def fire_write(g, b):
            base = (wid * n_ch + g - 1) * pkr
            pltpu.async_copy(pk_v.at[b // 2],
                             out_hbm.at[pl.ds(base, 2 * pkr)], wsem[b // 2])

        pltpu.sync_copy(comb_hbm.at[wid], comb_v)
        for b in range(rb):
            unpack_and_fire(b, b)

        def body(k, _):
            for b in range(rb):
                g = k * rb + b
                if b % 2 == 0:
                    @pl.when(k > 0)
                    def _():
                        wait_write(b, 2 * pkr)
                drain_pack(g, b)
                if b % 2 == 1:
                    fire_write(g, b)

                @pl.when(g + rb < n_ch)
                def _():
                    unpack_and_fire(g + rb, b)
            return 0

        lax.fori_loop(0, n_out, body, 0)
        for j in range(rem):
            g = n_out * rb + j
            if j % 2 == 0 and n_out > 0:
                wait_write(j, 2 * pkr)
            drain_pack(g, j)
            if j % 2 == 1:
                fire_write(g, j)
        if rem % 2 == 1:
            g = n_ch - 1
            pltpu.async_copy(pk_v.at[(rem - 1) // 2, pl.ds(0, pkr)],
                             out_hbm.at[pl.ds((wid * n_ch + g) * pkr, pkr)],
                             wsem[(rem - 1) // 2])
        last_slot_sizes = [2 * pkr, 2 * pkr]
        if rem % 2 == 1:
            last_slot_sizes[(rem - 1) // 2] = pkr
        for b2 in range(2):
            wait_write(2 * b2, last_slot_sizes[b2])

    return gat


def _pad_cols(a, width):
    pad = width - a.shape[1]
    if pad == 0:
        return a
    return jnp.concatenate([a, jnp.zeros((a.shape[0], pad), jnp.float32)],
                           axis=1)


def _pre1_kernel(x_ref, wl_ref, wr_ref, y_ref, r_ref):
    xb = x_ref[...]
    y = jnp.dot(xb, wl_ref[...], preferred_element_type=jnp.float32)
    ones = jnp.ones((xb.shape[0], WL - y.shape[1]), jnp.float32)
    y_ref[...] = jnp.concatenate([y, ones], axis=1)
    r_ref[...] = jnp.dot(xb, wr_ref[...], preferred_element_type=jnp.float32)


def _post_mid_body(ps, cnt, r, bl, g, b, wl_ref, wr_ref, y_ref, r2_ref):
    mean = ps / jnp.maximum(cnt, 1.0)
    t = mean + bl + r
    nrm = jnp.sqrt(jnp.sum(t * t, axis=-1, keepdims=True))
    t = t / jnp.maximum(nrm, 1e-12)
    mu = jnp.mean(t, axis=-1, keepdims=True)
    var = jnp.mean((t - mu) ** 2, axis=-1, keepdims=True)
    h = jnp.maximum((t - mu) / jnp.sqrt(var + 1e-5) * g + b, 0.0)
    y = jnp.dot(h, wl_ref[...], preferred_element_type=jnp.float32)
    y_ref[...] = _pad_cols(y, WL)
    r2_ref[...] = jnp.dot(h, wr_ref[...], preferred_element_type=jnp.float32)


def _post1_kernel(p_ref, r_ref, bl_ref, g_ref, be_ref, wl_ref, wr_ref,
                  y_ref, r2_ref, cnt_ref):
    ps = p_ref[0] + p_ref[1]
    cnt = ps[:, 64:65]
    cnt_ref[...] = cnt
    _post_mid_body(ps[:, :64], cnt, r_ref[...], bl_ref[...], g_ref[...],
                   be_ref[...], wl_ref, wr_ref, y_ref, r2_ref)


def _post2_kernel(p_ref, r_ref, bl_ref, g_ref, be_ref, wl_ref, wr_ref,
                  cnt_ref, y_ref, r2_ref):
    ps = p_ref[0] + p_ref[1]
    _post_mid_body(ps[:, :64], cnt_ref[...], r_ref[...], bl_ref[...],
                   g_ref[...], be_ref[...], wl_ref, wr_ref, y_ref, r2_ref)


def _post3_kernel(p_ref, r_ref, bl_ref, cnt_ref, h_ref):
    ps = p_ref[0] + p_ref[1]
    t = ps[:, :32] / jnp.maximum(cnt_ref[...], 1.0) + bl_ref[...] + r_ref[...]
    nrm = jnp.sqrt(jnp.sum(t * t, axis=-1, keepdims=True))
    h_ref[...] = _pad_cols(t / jnp.maximum(nrm, 1e-12), WL)


def _mlp_kernel(pk_ref, w1sd_ref, w1p_ref, w2_ref, w3_ref,
                b1_ref, b2_ref, b3_ref, o_ref):
    pk = pk_ref[...]
    pkb = pk.astype(jnp.bfloat16)
    prod = jnp.concatenate(
        [pk[:, 0:32] * pk[:, 32:64], pk[:, 64:96] * pk[:, 96:128]],
        axis=1).astype(jnp.bfloat16)
    z = (jnp.dot(pkb, w1sd_ref[...], preferred_element_type=jnp.float32)
         + jnp.dot(prod, w1p_ref[...], preferred_element_type=jnp.float32)
         + b1_ref[...])
    z = jnp.maximum(z, 0.0).astype(jnp.bfloat16)
    z = (jnp.dot(z, w2_ref[...], preferred_element_type=jnp.float32)
         + b2_ref[...])
    z = jnp.maximum(z, 0.0).astype(jnp.bfloat16)
    z = (jnp.dot(z, w3_ref[...], preferred_element_type=jnp.float32)
         + b3_ref[...])
    o_ref[...] = jax.nn.sigmoid(z)


def _full(shape):
    return pl.BlockSpec(shape, lambda i: tuple(0 for _ in shape))


def _rows(bs, w):
    return pl.BlockSpec((bs, w), lambda i: (i, 0))


def kernel(x, edge_index, W1l, b1l, W1r, W2l, b2l, W2r, W3l, b3l, W3r,
           g1, be1, g2, be2, mW1, mb1, mW2, mb2, mW3, mb3):
    n, d_in = x.shape
    e = edge_index.shape[1]
    h_dim = W1l.shape[1]
    out_dim = W3l.shape[1]
    src = edge_index[0].astype(jnp.int32)
    dst = edge_index[1].astype(jnp.int32)
    epw = e // NW
    comb3 = (src | (dst << 16)).reshape(NW, epw // CH, CH)
    npad = -(-n // (NS * ZR)) * (NS * ZR)

    bn = 1000
    gn = n // bn
    be_blk = 2560
    ge = e // be_blk

    y1, r1 = pl.pallas_call(
        _pre1_kernel,
        grid=(gn,),
        in_specs=[_rows(bn, d_in), _full((d_in, h_dim)), _full((d_in, h_dim))],
        out_specs=[_rows(bn, WL), _rows(bn, h_dim)],
        out_shape=[jax.ShapeDtypeStruct((n, WL), jnp.float32),
                   jax.ShapeDtypeStruct((n, h_dim), jnp.float32)],
    )(x, W1l, W1r)

    zrs = jnp.zeros((npad // NS, WL), jnp.float32)
    agg = _make_agg(n, e)
    p1 = agg(y1, comb3, zrs)

    y2, r2, cnt = pl.pallas_call(
        _post1_kernel,
        grid=(gn,),
        in_specs=[pl.BlockSpec((NC, bn, WL), lambda i: (0, i, 0)),
                  _rows(bn, h_dim), _full((1, h_dim)), _full((1, h_dim)),
                  _full((1, h_dim)), _full((h_dim, h_dim)),
                  _full((h_dim, h_dim))],
        out_specs=[_rows(bn, WL), _rows(bn, h_dim), _rows(bn, 1)],
        out_shape=[jax.ShapeDtypeStruct((n, WL), jnp.float32),
                   jax.ShapeDtypeStruct((n, h_dim), jnp.float32),
                   jax.ShapeDtypeStruct((n, 1), jnp.float32)],
    )(p1, r1, b1l.reshape(1, -1), g1.reshape(1, -1), be1.reshape(1, -1),
      W2l, W2r)

    p2 = agg(y2, comb3, zrs)
    y3, r3 = pl.pallas_call(
        _post2_kernel,
        grid=(gn,),
        in_specs=[pl.BlockSpec((NC, bn, WL), lambda i: (0, i, 0)),
                  _rows(bn, h_dim), _full((1, h_dim)), _full((1, h_dim)),
                  _full((1, h_dim)), _full((h_dim, out_dim)),
                  _full((h_dim, out_dim)), _rows(bn, 1)],
        out_specs=[_rows(bn, WL), _rows(bn, out_dim)],
        out_shape=[jax.ShapeDtypeStruct((n, WL), jnp.float32),
                   jax.ShapeDtypeStruct((n, out_dim), jnp.float32)],
    )(p2, r2, b2l.reshape(1, -1), g2.reshape(1, -1), be2.reshape(1, -1),
      W3l, W3r, cnt)

    p3 = agg(y3, comb3, zrs)
    h3 = pl.pallas_call(
        _post3_kernel,
        grid=(gn,),
        in_specs=[pl.BlockSpec((NC, bn, WL), lambda i: (0, i, 0)),
                  _rows(bn, out_dim), _full((1, out_dim)), _rows(bn, 1)],
        out_specs=_rows(bn, WL),
        out_shape=jax.ShapeDtypeStruct((n, WL), jnp.float32),
    )(p3, r3, b3l.reshape(1, -1), cnt)

    nch_w = epw // CH
    nch1 = nch_w * 3 // 5
    es1 = NW * CH * nch1
    comb_flat = comb3.reshape(-1)
    hsd1 = _make_edge_gather(n, es1, out_dim)(
        h3, comb_flat[:es1].reshape(NW, nch1, CH))
    hsd2 = _make_edge_gather(n, e - es1, out_dim)(
        h3, comb_flat[es1:].reshape(NW, nch_w - nch1, CH))
    bf = jnp.bfloat16
    w1sd = mW1[0:64]
    w1p = mW1[64:96]
    zz = jnp.zeros_like(w1sd)
    w1sd2 = jnp.concatenate(
        [jnp.concatenate([w1sd, zz], 1), jnp.concatenate([zz, w1sd], 1)],
        0).astype(bf)
    zp = jnp.zeros_like(w1p)
    w1p2 = jnp.concatenate(
        [jnp.concatenate([w1p, zp], 1), jnp.concatenate([zp, w1p], 1)],
        0).astype(bf)
    z2 = jnp.zeros_like(mW2)
    w2b = jnp.concatenate(
        [jnp.concatenate([mW2, z2], 1), jnp.concatenate([z2, mW2], 1)],
        0).astype(bf)
    z3 = jnp.zeros_like(mW3)
    w3b = jnp.concatenate(
        [jnp.concatenate([mW3, z3], 1), jnp.concatenate([z3, mW3], 1)],
        0).astype(bf)
    b1c = jnp.concatenate([mb1, mb1]).reshape(1, -1)
    b2c = jnp.concatenate([mb2, mb2]).reshape(1, -1)
    b3c = jnp.concatenate([mb3, mb3]).reshape(1, -1)
    bep = 3200

    def mlp(hsd):
        ne2 = hsd.shape[0]
        return pl.pallas_call(
            _mlp_kernel,
            grid=(ne2 // bep,),
            in_specs=[_rows(bep, WL),
                      _full((WL, WL)), _full((64, WL)), _full((WL, 64)),
                      _full((64, 2)), _full((1, WL)), _full((1, 64)),
                      _full((1, 2))],
            out_specs=_rows(bep, 2),
            out_shape=jax.ShapeDtypeStruct((ne2, 2), jnp.float32),
        )(hsd, w1sd2, w1p2, w2b, w3b, b1c, b2c, b3c)

    o1 = mlp(hsd1)
    o2 = mlp(hsd2)
    return jnp.concatenate([o1.reshape(es1), o2.reshape(e - es1)])


# speedup vs baseline: 1.0275x; 1.0275x over previous
import functools

import jax
import jax.numpy as jnp
from jax import lax
from jax.experimental import pallas as pl
from jax.experimental.pallas import tpu as pltpu
from jax.experimental.pallas import tpu_sc as plsc

NC = 2
NS = 16
NW = NC * NS
CH = 80
ZR = 128
WL = 128


def _make_agg(n, e):
    epw = e // NW
    n_ch = epw // CH
    rb = 3
    n_out = n_ch // rb
    rem = n_ch - n_out * rb
    npad = -(-n // (NS * ZR)) * (NS * ZR)
    rps = npad // NS
    mesh = plsc.VectorSubcoreMesh(core_axis_name="c", subcore_axis_name="s")

    @functools.partial(
        pl.kernel,
        mesh=mesh,
        out_type=jax.ShapeDtypeStruct((NC, npad, WL), jnp.float32),
        scratch_types=[
            pltpu.VMEM((n_ch, CH), jnp.int32),
            pltpu.VMEM((rb, CH), jnp.int32),
            pltpu.VMEM((rb, CH), jnp.int32),
            pltpu.VMEM((rb, CH, WL), jnp.float32),
            pltpu.VMEM_SHARED((npad, WL), jnp.float32),
            pltpu.SemaphoreType.DMA,
            pltpu.SemaphoreType.DMA,
            pltpu.SemaphoreType.DMA,
        ],
    )
    def agg(y_hbm, comb_hbm, zeros_hbm, out_hbm,
            comb_v, sidx_v, didx_v, rows_v, acc_sh, s0, s1, s2):
        sems = (s0, s1, s2)
        cid = lax.axis_index("c")
        sid = lax.axis_index("s")
        wid = sid * NC + cid

        def unpack_and_fire(g, b):
            for t in range(CH // 16):
                v = comb_v[g, pl.ds(t * 16, 16)]
                sidx_v[b, pl.ds(t * 16, 16)] = lax.bitwise_and(v, 0xFFFF)
                didx_v[b, pl.ds(t * 16, 16)] = lax.shift_right_logical(v, 16)
            pltpu.async_copy(y_hbm.at[sidx_v.at[b]], rows_v.at[b], sems[b])

        def drain_and_scatter(g, b):
            pltpu.make_async_copy(y_hbm.at[sidx_v.at[b]], rows_v.at[b],
                                  sems[b]).wait()
            pltpu.sync_copy(rows_v.at[b], acc_sh.at[didx_v.at[b]], add=True)

        pltpu.sync_copy(comb_hbm.at[wid], comb_v)
        pltpu.sync_copy(zeros_hbm, acc_sh.at[pl.ds(sid * rps, rps)])
        for b in range(rb):
            unpack_and_fire(b, b)
        plsc.subcore_barrier()

        def body(k, _):
            for b in range(rb):
                g = k * rb + b
                drain_and_scatter(g, b)

                @pl.when(g + rb < n_ch)
                def _():
                    unpack_and_fire(g + rb, b)
            return 0

        lax.fori_loop(0, n_out, body, 0)
        for j in range(rem):
            drain_and_scatter(n_out * rb + j, j)
        plsc.subcore_barrier()

        pltpu.sync_copy(acc_sh.at[pl.ds(sid * rps, rps)],
                        out_hbm.at[cid, pl.ds(sid * rps, rps)])

    return agg


def _make_edge_gather(n, e, w):
    epw = e // NW
    n_ch = epw // CH
    rb = 4
    n_out = n_ch // rb
    rem = n_ch - n_out * rb
    pkr = CH // 2
    mesh = plsc.VectorSubcoreMesh(core_axis_name="c", subcore_axis_name="s")

    @functools.partial(
        pl.kernel,
        mesh=mesh,
        out_type=jax.ShapeDtypeStruct((e // 2, WL), jnp.float32),
        scratch_types=[
            pltpu.VMEM((n_ch, CH), jnp.int32),
            pltpu.VMEM((rb, CH), jnp.int32),
            pltpu.VMEM((rb, CH), jnp.int32),
            pltpu.VMEM((rb, CH, WL), jnp.float32),
            pltpu.VMEM((rb, CH, WL), jnp.float32),
            pltpu.VMEM((rb // 2, 2 * pkr, WL), jnp.float32),
            pltpu.SemaphoreType.DMA,
            pltpu.SemaphoreType.DMA,
            pltpu.SemaphoreType.DMA,
            pltpu.SemaphoreType.DMA,
            pltpu.SemaphoreType.DMA,
            pltpu.SemaphoreType.DMA,
        ],
    )
    def gat(h_hbm, comb_hbm, out_hbm, comb_v, sidx_v, didx_v,
            srows_v, drows_v, pk_v, g0, g1, g2, g3, w0, w1):
        gsem = (g0, g1, g2, g3)
        wsem = (w0, w1)
        cid = lax.axis_index("c")
        sid = lax.axis_index("s")
        wid = sid * NC + cid

        def unpack_and_fire(g, b):
            for t in range(CH // 16):
                v = comb_v[g, pl.ds(t * 16, 16)]
                sidx_v[b, pl.ds(t * 16, 16)] = lax.bitwise_and(v, 0xFFFF)
                didx_v[b, pl.ds(t * 16, 16)] = lax.shift_right_logical(v, 16)
            pltpu.async_copy(h_hbm.at[sidx_v.at[b]], srows_v.at[b], gsem[b])
            pltpu.async_copy(h_hbm.at[didx_v.at[b]], drows_v.at[b], gsem[b])

        def wait_write(b, nrows):
            pltpu.make_async_copy(pk_v.at[b // 2, pl.ds(0, nrows)],
                                  out_hbm.at[pl.ds(0, nrows)], wsem[b // 2]
                                  ).wait()

        def drain_pack(g, b):
            pltpu.make_async_copy(h_hbm.at[sidx_v.at[b]], srows_v.at[b],
                                  gsem[b]).wait()
            pltpu.make_async_copy(h_hbm.at[didx_v.at[b]], drows_v.at[b],
                                  gsem[b]).wait()
            half = (b % 2) * pkr
            for j in range(pkr):
                for q, (ref, row) in enumerate(
                        ((srows_v, 2 * j), (drows_v, 2 * j),
                         (srows_v, 2 * j + 1), (drows_v, 2 * j + 1))):
                    for t in range(2):
                        pk_v[b // 2, half + j, pl.ds(32 * q + 16 * t, 16)] = (
                            ref[b, row, pl.ds(16 * t, 16)])

        def fire_write(g, b):
            base = (wid * n_ch + g - 1) * pkr
            pltpu.async_copy(pk_v.at[b // 2],
                             out_hbm.at[pl.ds(base, 2 * pkr)], wsem[b // 2])

        pltpu.sync_copy(comb_hbm.at[wid], comb_v)
        for b in range(rb):
            unpack_and_fire(b, b)

        def body(k, _):
            for b in range(rb):
                g = k * rb + b
                if b % 2 == 0:
                    @pl.when(k > 0)
                    def _():
                        wait_write(b, 2 * pkr)
                drain_pack(g, b)
                if b % 2 == 1:
                    fire_write(g, b)

                @pl.when(g + rb < n_ch)
                def _():
                    unpack_and_fire(g + rb, b)
            return 0

        lax.fori_loop(0, n_out, body, 0)
        for j in range(rem):
            g = n_out * rb + j
            if j % 2 == 0 and n_out > 0:
                wait_write(j, 2 * pkr)
            drain_pack(g, j)
            if j % 2 == 1:
                fire_write(g, j)
        if rem % 2 == 1:
            g = n_ch - 1
            pltpu.async_copy(pk_v.at[(rem - 1) // 2, pl.ds(0, pkr)],
                             out_hbm.at[pl.ds((wid * n_ch + g) * pkr, pkr)],
                             wsem[(rem - 1) // 2])
        last_slot_sizes = [2 * pkr, 2 * pkr]
        if rem % 2 == 1:
            last_slot_sizes[(rem - 1) // 2] = pkr
        for b2 in range(2):
            wait_write(2 * b2, last_slot_sizes[b2])

    return gat


def _pad_cols(a, width):
    pad = width - a.shape[1]
    if pad == 0:
        return a
    return jnp.concatenate([a, jnp.zeros((a.shape[0], pad), jnp.float32)],
                           axis=1)


def _pre1_kernel(x_ref, wl_ref, wr_ref, y_ref, r_ref):
    xb = x_ref[...]
    y = jnp.dot(xb, wl_ref[...], preferred_element_type=jnp.float32)
    ones = jnp.ones((xb.shape[0], WL - y.shape[1]), jnp.float32)
    y_ref[...] = jnp.concatenate([y, ones], axis=1)
    r_ref[...] = jnp.dot(xb, wr_ref[...], preferred_element_type=jnp.float32)


def _post_mid_body(ps, cnt, r, bl, g, b, wl_ref, wr_ref, y_ref, r2_ref):
    mean = ps / jnp.maximum(cnt, 1.0)
    t = mean + bl + r
    nrm = jnp.sqrt(jnp.sum(t * t, axis=-1, keepdims=True))
    t = t / jnp.maximum(nrm, 1e-12)
    mu = jnp.mean(t, axis=-1, keepdims=True)
    var = jnp.mean((t - mu) ** 2, axis=-1, keepdims=True)
    h = jnp.maximum((t - mu) / jnp.sqrt(var + 1e-5) * g + b, 0.0)
    y = jnp.dot(h, wl_ref[...], preferred_element_type=jnp.float32)
    y_ref[...] = _pad_cols(y, WL)
    r2_ref[...] = jnp.dot(h, wr_ref[...], preferred_element_type=jnp.float32)


def _post1_kernel(p_ref, r_ref, bl_ref, g_ref, be_ref, wl_ref, wr_ref,
                  y_ref, r2_ref, cnt_ref):
    ps = p_ref[0] + p_ref[1]
    cnt = ps[:, 64:65]
    cnt_ref[...] = cnt
    _post_mid_body(ps[:, :64], cnt, r_ref[...], bl_ref[...], g_ref[...],
                   be_ref[...], wl_ref, wr_ref, y_ref, r2_ref)


def _post2_kernel(p_ref, r_ref, bl_ref, g_ref, be_ref, wl_ref, wr_ref,
                  cnt_ref, y_ref, r2_ref):
    ps = p_ref[0] + p_ref[1]
    _post_mid_body(ps[:, :64], cnt_ref[...], r_ref[...], bl_ref[...],
                   g_ref[...], be_ref[...], wl_ref, wr_ref, y_ref, r2_ref)


def _post3_kernel(p_ref, r_ref, bl_ref, cnt_ref, h_ref):
    ps = p_ref[0] + p_ref[1]
    t = ps[:, :32] / jnp.maximum(cnt_ref[...], 1.0) + bl_ref[...] + r_ref[...]
    nrm = jnp.sqrt(jnp.sum(t * t, axis=-1, keepdims=True))
    h_ref[...] = _pad_cols(t / jnp.maximum(nrm, 1e-12), WL)


def _mlp_kernel(pk_ref, w1sd_ref, w1p_ref, w2_ref, w3_ref,
                b1_ref, b2_ref, b3_ref, o_ref):
    pk = pk_ref[...]
    pkb = pk.astype(jnp.bfloat16)
    prod = jnp.concatenate(
        [pk[:, 0:32] * pk[:, 32:64], pk[:, 64:96] * pk[:, 96:128]],
        axis=1).astype(jnp.bfloat16)
    z = (jnp.dot(pkb, w1sd_ref[...], preferred_element_type=jnp.float32)
         + jnp.dot(prod, w1p_ref[...], preferred_element_type=jnp.float32)
         + b1_ref[...])
    z = jnp.maximum(z, 0.0).astype(jnp.bfloat16)
    z = (jnp.dot(z, w2_ref[...], preferred_element_type=jnp.float32)
         + b2_ref[...])
    z = jnp.maximum(z, 0.0).astype(jnp.bfloat16)
    z = (jnp.dot(z, w3_ref[...], preferred_element_type=jnp.float32)
         + b3_ref[...])
    o_ref[...] = jax.nn.sigmoid(z)


def _full(shape):
    return pl.BlockSpec(shape, lambda i: tuple(0 for _ in shape))


def _rows(bs, w):
    return pl.BlockSpec((bs, w), lambda i: (i, 0))


def kernel(x, edge_index, W1l, b1l, W1r, W2l, b2l, W2r, W3l, b3l, W3r,
           g1, be1, g2, be2, mW1, mb1, mW2, mb2, mW3, mb3):
    n, d_in = x.shape
    e = edge_index.shape[1]
    h_dim = W1l.shape[1]
    out_dim = W3l.shape[1]
    src = edge_index[0].astype(jnp.int32)
    dst = edge_index[1].astype(jnp.int32)
    epw = e // NW
    comb3 = (src | (dst << 16)).reshape(NW, epw // CH, CH)
    npad = -(-n // (NS * ZR)) * (NS * ZR)

    bn = 1000
    gn = n // bn
    be_blk = 2560
    ge = e // be_blk

    y1, r1 = pl.pallas_call(
        _pre1_kernel,
        grid=(gn,),
        in_specs=[_rows(bn, d_in), _full((d_in, h_dim)), _full((d_in, h_dim))],
        out_specs=[_rows(bn, WL), _rows(bn, h_dim)],
        out_shape=[jax.ShapeDtypeStruct((n, WL), jnp.float32),
                   jax.ShapeDtypeStruct((n, h_dim), jnp.float32)],
    )(x, W1l, W1r)

    zrs = jnp.zeros((npad // NS, WL), jnp.float32)
    agg = _make_agg(n, e)
    p1 = agg(y1, comb3, zrs)

    y2, r2, cnt = pl.pallas_call(
        _post1_kernel,
        grid=(gn,),
        in_specs=[pl.BlockSpec((NC, bn, WL), lambda i: (0, i, 0)),
                  _rows(bn, h_dim), _full((1, h_dim)), _full((1, h_dim)),
                  _full((1, h_dim)), _full((h_dim, h_dim)),
                  _full((h_dim, h_dim))],
        out_specs=[_rows(bn, WL), _rows(bn, h_dim), _rows(bn, 1)],
        out_shape=[jax.ShapeDtypeStruct((n, WL), jnp.float32),
                   jax.ShapeDtypeStruct((n, h_dim), jnp.float32),
                   jax.ShapeDtypeStruct((n, 1), jnp.float32)],
    )(p1, r1, b1l.reshape(1, -1), g1.reshape(1, -1), be1.reshape(1, -1),
      W2l, W2r)

    p2 = agg(y2, comb3, zrs)
    y3, r3 = pl.pallas_call(
        _post2_kernel,
        grid=(gn,),
        in_specs=[pl.BlockSpec((NC, bn, WL), lambda i: (0, i, 0)),
                  _rows(bn, h_dim), _full((1, h_dim)), _full((1, h_dim)),
                  _full((1, h_dim)), _full((h_dim, out_dim)),
                  _full((h_dim, out_dim)), _rows(bn, 1)],
        out_specs=[_rows(bn, WL), _rows(bn, out_dim)],
        out_shape=[jax.ShapeDtypeStruct((n, WL), jnp.float32),
                   jax.ShapeDtypeStruct((n, out_dim), jnp.float32)],
    )(p2, r2, b2l.reshape(1, -1), g2.reshape(1, -1), be2.reshape(1, -1),
      W3l, W3r, cnt)

    p3 = agg(y3, comb3, zrs)
    h3 = pl.pallas_call(
        _post3_kernel,
        grid=(gn,),
        in_specs=[pl.BlockSpec((NC, bn, WL), lambda i: (0, i, 0)),
                  _rows(bn, out_dim), _full((1, out_dim)), _rows(bn, 1)],
        out_specs=_rows(bn, WL),
        out_shape=jax.ShapeDtypeStruct((n, WL), jnp.float32),
    )(p3, r3, b3l.reshape(1, -1), cnt)

    nch_w = epw // CH
    nch1 = nch_w * 3 // 5
    es1 = NW * CH * nch1
    comb_flat = comb3.reshape(-1)
    hsd1 = _make_edge_gather(n, es1, out_dim)(
        h3, comb_flat[:es1].reshape(NW, nch1, CH))
    hsd2 = _make_edge_gather(n, e - es1, out_dim)(
        h3, comb_flat[es1:].reshape(NW, nch_w - nch1, CH))
    bf = jnp.bfloat16
    w1sd = mW1[0:64]
    w1p = mW1[64:96]
    zz = jnp.zeros_like(w1sd)
    w1sd2 = jnp.concatenate(
        [jnp.concatenate([w1sd, zz], 1), jnp.concatenate([zz, w1sd], 1)],
        0).astype(bf)
    zp = jnp.zeros_like(w1p)
    w1p2 = jnp.concatenate(
        [jnp.concatenate([w1p, zp], 1), jnp.concatenate([zp, w1p], 1)],
        0).astype(bf)
    z2 = jnp.zeros_like(mW2)
    w2b = jnp.concatenate(
        [jnp.concatenate([mW2, z2], 1), jnp.concatenate([z2, mW2], 1)],
        0).astype(bf)
    z3 = jnp.zeros_like(mW3)
    w3b = jnp.concatenate(
        [jnp.concatenate([mW3, z3], 1), jnp.concatenate([z3, mW3], 1)],
        0).astype(bf)
    b1c = jnp.concatenate([mb1, mb1]).reshape(1, -1)
    b2c = jnp.concatenate([mb2, mb2]).reshape(1, -1)
    b3c = jnp.concatenate([mb3, mb3]).reshape(1, -1)
    bep = 3200

    def mlp(hsd):
        ne2 = hsd.shape[0]
        return pl.pallas_call(
            _mlp_kernel,
            grid=(ne2 // bep,),
            in_specs=[_rows(bep, WL),
                      _full((WL, WL)), _full((64, WL)), _full((WL, 64)),
                      _full((64, 2)), _full((1, WL)), _full((1, 64)),
                      _full((1, 2))],
            out_specs=_rows(bep, 2),
            out_shape=jax.ShapeDtypeStruct((ne2, 2), jnp.float32),
        )(hsd, w1sd2, w1p2, w2b, w3b, b1c, b2c, b3c)

    o1 = mlp(hsd1)
    o2 = mlp(hsd2)
    return jnp.concatenate([o1.reshape(es1), o2.reshape(e - es1)])
